# baseline (speedup 1.0000x reference)
import functools

import jax
import jax.numpy as jnp
from jax import lax
from jax.experimental import pallas as pl
from jax.experimental.pallas import tpu as pltpu
from jax.experimental.pallas import tpu_sc as plsc

_NF = 26
_V = 100
_D = 64
_B = 16384
_NP = _NF // 2
_PD = 2 * _D
_NW = 32
_BPW = _B // _NW
_CB = 32
_NCH = _BPW // _CB


def _build_pair_table(Wf, idx_t):
    def body(w_ref, i_ref, out_ref, idxp_ref):
        wl = w_ref[0]
        wr = w_ref[1]
        left = jnp.broadcast_to(wl[:, None, :], (_V, _V, _D)).reshape(_V * _V, _D)
        right = jnp.broadcast_to(wr[None, :, :], (_V, _V, _D)).reshape(_V * _V, _D)
        out_ref[...] = jnp.concatenate([left, right], axis=1)
        j = pl.program_id(0)
        idxp_ref[...] = (i_ref[0, 0] * _V + i_ref[0, 1] + j * (_V * _V))[None, None]

    return pl.pallas_call(
        body,
        grid=(_NP,),
        in_specs=[
            pl.BlockSpec((2, _V, _D), lambda j: (j, 0, 0)),
            pl.BlockSpec((1, 2, _B), lambda j: (j, 0, 0)),
        ],
        out_specs=[
            pl.BlockSpec((_V * _V, _PD), lambda j: (j, 0)),
            pl.BlockSpec((1, 1, _B), lambda j: (j, 0, 0)),
        ],
        out_shape=[
            jax.ShapeDtypeStruct((_NP * _V * _V, _PD), jnp.float32),
            jax.ShapeDtypeStruct((_NP, 1, _B), jnp.int32),
        ],
    )(Wf, idx_t)


def _sc_gather(idxp, tablep):
    mesh = plsc.VectorSubcoreMesh(core_axis_name="c", subcore_axis_name="s")

    @functools.partial(
        pl.kernel,
        mesh=mesh,
        out_type=jax.ShapeDtypeStruct((_B, _NP * _PD), jnp.float32),
        scratch_types=[
            pltpu.VMEM((_NP, _BPW), jnp.int32),
            pltpu.VMEM((_CB, _NP * _PD), jnp.float32),
            pltpu.VMEM((_CB, _NP * _PD), jnp.float32),
            pltpu.SemaphoreType.DMA,
            pltpu.SemaphoreType.DMA,
            pltpu.SemaphoreType.DMA,
        ],
    )
    def k(idx_hbm, table_hbm, out_hbm, idxbuf, rbuf0, rbuf1, isem, sem0, sem1):
        w = lax.axis_index("s") * 2 + lax.axis_index("c")
        bbase = w * _BPW

        pltpu.async_copy(
            idx_hbm.at[:, 0, pl.ds(bbase, _BPW)], idxbuf, isem
        ).wait()

        def fire(c, rbuf, sem):
            for j in range(_NP):
                pltpu.async_copy(
                    table_hbm.at[idxbuf.at[j, pl.ds(c * _CB, _CB)]],
                    rbuf.at[:, pl.ds(j * _PD, _PD)],
                    sem,
                )

        def drain(rbuf, sem):
            pltpu.make_async_copy(out_hbm.at[pl.ds(0, _CB)], rbuf, sem).wait()

        fire(0, rbuf0, sem0)

        def pair(i, carry):
            a = 2 * i
            fire(a + 1, rbuf1, sem1)
            drain(rbuf0, sem0)
            pltpu.sync_copy(rbuf0, out_hbm.at[pl.ds(bbase + a * _CB, _CB)])
            fire(lax.min(a + 2, _NCH - 1), rbuf0, sem0)
            drain(rbuf1, sem1)
            pltpu.sync_copy(rbuf1, out_hbm.at[pl.ds(bbase + (a + 1) * _CB, _CB)])
            return carry

        lax.fori_loop(0, _NCH // 2, pair, 0)
        drain(rbuf0, sem0)

    return k(idxp, tablep)


def kernel(index_sentences, W):
    idx_t = index_sentences.astype(jnp.int32).T.reshape(_NP, 2, _B)
    tablep, idxp = _build_pair_table(W.astype(jnp.float32), idx_t)
    return _sc_gather(idxp, tablep)


# speedup vs baseline: 1.0158x; 1.0158x over previous
import functools

import jax
import jax.numpy as jnp
from jax import lax
from jax.experimental import pallas as pl
from jax.experimental.pallas import tpu as pltpu
from jax.experimental.pallas import tpu_sc as plsc

_NF = 26
_V = 100
_D = 64
_B = 16384
_NP = _NF // 2
_PD = 2 * _D
_NW = 32
_BPW = _B // _NW
_CB = 16
_NCH = _BPW // _CB


def _build_pair_table(Wf, idx_t):
    def body(w_ref, i_ref, out_ref, idxp_ref):
        wl = w_ref[0]
        wr = w_ref[1]
        left = jnp.broadcast_to(wl[:, None, :], (_V, _V, _D)).reshape(_V * _V, _D)
        right = jnp.broadcast_to(wr[None, :, :], (_V, _V, _D)).reshape(_V * _V, _D)
        out_ref[...] = jnp.concatenate([left, right], axis=1)
        j = pl.program_id(0)
        idxp_ref[...] = (i_ref[0, 0] * _V + i_ref[0, 1] + j * (_V * _V))[None, None]

    return pl.pallas_call(
        body,
        grid=(_NP,),
        in_specs=[
            pl.BlockSpec((2, _V, _D), lambda j: (j, 0, 0)),
            pl.BlockSpec((1, 2, _B), lambda j: (j, 0, 0)),
        ],
        out_specs=[
            pl.BlockSpec((_V * _V, _PD), lambda j: (j, 0)),
            pl.BlockSpec((1, 1, _B), lambda j: (j, 0, 0)),
        ],
        out_shape=[
            jax.ShapeDtypeStruct((_NP * _V * _V, _PD), jnp.float32),
            jax.ShapeDtypeStruct((_NP, 1, _B), jnp.int32),
        ],
    )(Wf, idx_t)


def _sc_gather(idxp, tablep):
    mesh = plsc.VectorSubcoreMesh(core_axis_name="c", subcore_axis_name="s")

    @functools.partial(
        pl.kernel,
        mesh=mesh,
        out_type=jax.ShapeDtypeStruct((_B, _NP * _PD), jnp.float32),
        scratch_types=[
            pltpu.VMEM((_NP, _BPW), jnp.int32),
            pltpu.VMEM((_CB, _NP * _PD), jnp.float32),
            pltpu.VMEM((_CB, _NP * _PD), jnp.float32),
            pltpu.SemaphoreType.DMA,
            pltpu.SemaphoreType.DMA,
            pltpu.SemaphoreType.DMA,
        ],
    )
    def k(idx_hbm, table_hbm, out_hbm, idxbuf, rbuf0, rbuf1, isem, sem0, sem1):
        w = lax.axis_index("s") * 2 + lax.axis_index("c")
        bbase = w * _BPW

        pltpu.async_copy(
            idx_hbm.at[:, 0, pl.ds(bbase, _BPW)], idxbuf, isem
        ).wait()

        def fire(c, rbuf, sem):
            for j in range(_NP):
                pltpu.async_copy(
                    table_hbm.at[idxbuf.at[j, pl.ds(c * _CB, _CB)]],
                    rbuf.at[:, pl.ds(j * _PD, _PD)],
                    sem,
                )

        def drain(rbuf, sem):
            pltpu.make_async_copy(out_hbm.at[pl.ds(0, _CB)], rbuf, sem).wait()

        fire(0, rbuf0, sem0)

        def pair(i, carry):
            a = 2 * i
            fire(a + 1, rbuf1, sem1)
            drain(rbuf0, sem0)
            pltpu.sync_copy(rbuf0, out_hbm.at[pl.ds(bbase + a * _CB, _CB)])
            fire(lax.min(a + 2, _NCH - 1), rbuf0, sem0)
            drain(rbuf1, sem1)
            pltpu.sync_copy(rbuf1, out_hbm.at[pl.ds(bbase + (a + 1) * _CB, _CB)])
            return carry

        lax.fori_loop(0, _NCH // 2, pair, 0)
        drain(rbuf0, sem0)

    return k(idxp, tablep)


def kernel(index_sentences, W):
    idx_t = index_sentences.astype(jnp.int32).T.reshape(_NP, 2, _B)
    tablep, idxp = _build_pair_table(W.astype(jnp.float32), idx_t)
    return _sc_gather(idxp, tablep)
